# Initial kernel scaffold; baseline (speedup 1.0000x reference)
import jax
import jax.numpy as jnp
from jax.experimental import pallas as pl


def kernel(emb, pos, sid, mod, role, padding_mask, proj_W, proj_b, cls_content, pos_embed, id_embed, mod_embed, role_embed):
    raise NotImplementedError("write your pallas kernel here")


# speedup vs baseline: 5.2686x; 5.2686x over previous
import functools

import jax
import jax.numpy as jnp
from jax import lax
from jax.experimental import pallas as pl
from jax.experimental.pallas import tpu as pltpu
from jax.experimental.pallas import tpu_sc as plsc


def _pos_gather_sc(pos_embed, idx, n_tokens, d_model):
    info = plsc.get_sparse_core_info()
    nw = info.num_cores * info.num_subcores
    chunk = n_tokens // nw
    sub = min(chunk, 64)
    steps = chunk // sub

    mesh = plsc.VectorSubcoreMesh(core_axis_name="c", subcore_axis_name="s")

    @functools.partial(
        pl.kernel,
        mesh=mesh,
        out_type=jax.ShapeDtypeStruct((n_tokens, d_model), jnp.float32),
        scratch_types=[
            pltpu.VMEM((sub,), jnp.int32),
            pltpu.VMEM((sub, d_model), jnp.float32),
            pltpu.SemaphoreType.DMA,
        ],
    )
    def gather_kernel(table_hbm, idx_hbm, out_hbm, idx_v, rows_v, sem):
        wid = lax.axis_index("s") * info.num_cores + lax.axis_index("c")
        base = wid * chunk
        for h in range(steps):
            off = base + h * sub
            pltpu.sync_copy(idx_hbm.at[pl.ds(off, sub)], idx_v)
            pltpu.async_copy(table_hbm.at[idx_v], rows_v, sem).wait()
            pltpu.sync_copy(rows_v, out_hbm.at[pl.ds(off, sub)])

    return gather_kernel(pos_embed, idx)


def _encode_tc_body(emb_ref, meta_ref, w_ref, t_ref, g_ref, out_ref, *, blk, d_in, s, dm):
    meta = meta_ref[0]
    sid = meta[:, 0:1]
    mod = meta[:, 1:2]
    role = meta[:, 2:3]
    msk = meta[:, 3:4]

    emb = emb_ref[...]
    emb_t = jnp.concatenate([emb] * s, axis=1)
    colex = lax.broadcasted_iota(jnp.int32, (blk, s * d_in), 1) // d_in
    keep = (colex == sid) & (msk != 0)
    x2 = jnp.where(keep, emb_t, jnp.bfloat16(0))
    acc = jnp.dot(x2, w_ref[...], preferred_element_type=jnp.float32)

    cols = lax.broadcasted_iota(jnp.int32, (blk, 2 * s + 8), 1)
    one = jnp.float32(1)
    zero = jnp.float32(0)
    oh = jnp.where(
        cols < s, jnp.where((cols == sid) & (msk != 0), one, zero),
        jnp.where(cols < 2 * s, jnp.where(cols - s == sid, one, zero),
                  jnp.where(cols < 2 * s + 4,
                            jnp.where(cols - 2 * s == mod, one, zero),
                            jnp.where(cols - (2 * s + 4) == role, one, zero))))
    out_ref[...] = acc + jnp.dot(oh, t_ref[...], preferred_element_type=jnp.float32) + g_ref[...]


def kernel(emb, pos, sid, mod, role, padding_mask, proj_W, proj_b,
           cls_content, pos_embed, id_embed, mod_embed, role_embed):
    b, l, d_in = emb.shape
    s, _, dm = proj_W.shape
    n = b * l
    blk = 512
    nb = n // blk

    g = _pos_gather_sc(pos_embed, pos.reshape(n).astype(jnp.int32), n, dm)

    emb_bf = emb.reshape(n, d_in).astype(jnp.bfloat16)
    w_flat = proj_W.reshape(s * d_in, dm).astype(jnp.bfloat16)
    trows = 2 * s + 8
    t_tab = jnp.concatenate(
        [proj_b, id_embed[:s], mod_embed, role_embed,
         jnp.zeros((trows - 2 * s - mod_embed.shape[0] - role_embed.shape[0], dm),
                   jnp.float32)], axis=0)
    meta = jnp.stack(
        [sid.reshape(n), mod.reshape(n), role.reshape(n),
         padding_mask.reshape(n).astype(jnp.int32)], axis=-1).reshape(nb, blk, 4)

    tok = pl.pallas_call(
        functools.partial(_encode_tc_body, blk=blk, d_in=d_in, s=s, dm=dm),
        grid=(nb,),
        in_specs=[
            pl.BlockSpec((blk, d_in), lambda i: (i, 0)),
            pl.BlockSpec((1, blk, 4), lambda i: (i, 0, 0)),
            pl.BlockSpec((s * d_in, dm), lambda i: (0, 0)),
            pl.BlockSpec((trows, dm), lambda i: (0, 0)),
            pl.BlockSpec((blk, dm), lambda i: (i, 0)),
        ],
        out_specs=pl.BlockSpec((blk, dm), lambda i: (i, 0)),
        out_shape=jax.ShapeDtypeStruct((n, dm), jnp.float32),
    )(emb_bf, meta, w_flat, t_tab, g)

    cls_row = cls_content + pos_embed[0] + id_embed[s]
    tokens = jnp.concatenate(
        [jnp.broadcast_to(cls_row, (b, 1, dm)), tok.reshape(b, l, dm)], axis=1)
    attn_keep = jnp.concatenate(
        [jnp.ones((b, 1), dtype=bool), padding_mask], axis=1)
    return tokens, attn_keep
